# baseline (speedup 1.0000x reference)
import functools

import jax
import jax.numpy as jnp
from jax import lax
from jax.experimental import pallas as pl
from jax.experimental.pallas import tpu as pltpu
from jax.experimental.pallas import tpu_sc as plsc

N = 16384
L = 16
NS = 16
CHUNK = N // NS
IMASK = 0x7FFFFFFF
MSB = -(2**31)


def _f32key(x):
    i = plsc.bitcast(x, jnp.int32)
    return jnp.where(i >= 0, i, i ^ IMASK)


def _unkey_sq(kv):
    iv = jnp.where(kv >= 0, kv, kv ^ IMASK)
    xv = plsc.bitcast(iv, jnp.float32)
    return xv * xv


@functools.partial(
    pl.kernel,
    out_type=jax.ShapeDtypeStruct((L,), jnp.float32),
    mesh=plsc.VectorSubcoreMesh(core_axis_name="c", subcore_axis_name="s",
                                num_cores=1),
    compiler_params=pltpu.CompilerParams(needs_layout_passes=False,
                                         skip_device_barrier=True),
    scratch_types=[
        pltpu.VMEM((CHUNK,), jnp.float32),
        pltpu.VMEM((CHUNK,), jnp.float32),
        pltpu.VMEM((CHUNK,), jnp.float32),
        pltpu.VMEM((5 * L,), jnp.float32),
        pltpu.VMEM_SHARED((NS * 5 * L,), jnp.float32),
        pltpu.VMEM((NS * 5 * L,), jnp.float32),
        pltpu.VMEM((N,), jnp.float32),
        pltpu.VMEM((N,), jnp.float32),
        pltpu.VMEM((N,), jnp.float32),
        pltpu.VMEM((N,), jnp.int32),
        pltpu.VMEM((L,), jnp.float32),
        pltpu.SemaphoreType.DMA,
        pltpu.SemaphoreType.DMA,
        pltpu.SemaphoreType.DMA,
    ],
)
def _sc_loss(p0_hbm, p1_hbm, gt_hbm, out_hbm,
             gt_v, p0_v, p1_v, part_v, part_sh, red_v,
             fgt_v, fp0_v, fp1_v, keys_v, res_v, sem0, sem1, sem2):
    c = lax.axis_index("c")
    s = lax.axis_index("s")
    base = s * CHUNK

    HALF = CHUNK // 2
    cp0 = pltpu.async_copy(gt_hbm.at[pl.ds(base, HALF)], gt_v.at[pl.ds(0, HALF)], sem0)
    cp1 = pltpu.async_copy(p0_hbm.at[pl.ds(base, HALF)], p0_v.at[pl.ds(0, HALF)], sem1)
    cp2 = pltpu.async_copy(p1_hbm.at[pl.ds(base, HALF)], p1_v.at[pl.ds(0, HALF)], sem2)
    cp3 = pltpu.async_copy(gt_hbm.at[pl.ds(base + HALF, HALF)], gt_v.at[pl.ds(HALF, HALF)], sem0)
    cp4 = pltpu.async_copy(p0_hbm.at[pl.ds(base + HALF, HALF)], p0_v.at[pl.ds(HALF, HALF)], sem1)
    cp5 = pltpu.async_copy(p1_hbm.at[pl.ds(base + HALF, HALF)], p1_v.at[pl.ds(HALF, HALF)], sem2)

    zf = jnp.zeros((L,), jnp.float32)

    def p1_step(i, carry):
        cnt, pt0, pt1, sq0, sq1 = carry
        for u in range(2):
            off = (i * 2 + u) * L
            g = gt_v[pl.ds(off, L)]
            a = p0_v[pl.ds(off, L)]
            b = p1_v[pl.ds(off, L)]
            posf = jnp.where(g == 1.0, jnp.float32(1.0), jnp.float32(0.0))
            negf = jnp.float32(1.0) - posf
            da = a - jnp.float32(1.0)
            db = b - jnp.float32(1.0)
            na = a * negf
            nb = b * negf
            cnt = cnt + posf
            pt0 = pt0 + da * da * posf
            pt1 = pt1 + db * db * posf
            sq0 = sq0 + na * na
            sq1 = sq1 + nb * nb
        return (cnt, pt0, pt1, sq0, sq1)

    NH = HALF // L // 2
    cp0.wait()
    cp1.wait()
    cp2.wait()
    acc = lax.fori_loop(0, NH, p1_step, (zf, zf, zf, zf, zf))
    cp3.wait()
    cp4.wait()
    cp5.wait()
    cnt, pt0, pt1, sq0, sq1 = lax.fori_loop(NH, 2 * NH, p1_step, acc)

    part_v[pl.ds(0 * L, L)] = cnt
    part_v[pl.ds(1 * L, L)] = pt0
    part_v[pl.ds(2 * L, L)] = pt1
    part_v[pl.ds(3 * L, L)] = sq0
    part_v[pl.ds(4 * L, L)] = sq1
    pltpu.sync_copy(part_v, part_sh.at[pl.ds(s * 5 * L, 5 * L)])
    plsc.subcore_barrier()

    @pl.when(s == 0)
    def _phase2():
        pltpu.sync_copy(part_sh, red_v)

        def red_step(i, carry):
            tcnt, tpt, tsq0, tsq1 = carry
            row = i * 5 * L
            tcnt = tcnt + red_v[pl.ds(row + 0 * L, L)]
            tpt = (tpt + red_v[pl.ds(row + 1 * L, L)]
                   + red_v[pl.ds(row + 2 * L, L)])
            tsq0 = tsq0 + red_v[pl.ds(row + 3 * L, L)]
            tsq1 = tsq1 + red_v[pl.ds(row + 4 * L, L)]
            return tcnt, tpt, tsq0, tsq1

        tcnt, tpt, tsq0, tsq1 = lax.fori_loop(
            0, NS, red_step, (zf, zf, zf, zf))

        num_pos = jnp.sum(tcnt)
        num_pos_i = num_pos.astype(jnp.int32)
        k = jnp.minimum(num_pos_i * 3, N)
        npd = jnp.maximum(num_pos, jnp.float32(1.0))
        pt_raw = jnp.sum(tpt)

        def fast_nopt():
            return jnp.sum(tsq0) + jnp.sum(tsq1)

        def slow_nopt():
            pltpu.sync_copy(gt_hbm.at[pl.ds(0, N)], fgt_v)
            pltpu.sync_copy(p0_hbm.at[pl.ds(0, N)], fp0_v)
            pltpu.sync_copy(p1_hbm.at[pl.ds(0, N)], fp1_v)

            def per_stack(pred_ref):
                def key_step(i, carry):
                    g = fgt_v[pl.ds(i * L, L)]
                    p = pred_ref[pl.ds(i * L, L)]
                    negf = jnp.where(g == 1.0, jnp.float32(0.0),
                                     jnp.float32(1.0))
                    keys_v[pl.ds(i * L, L)] = _f32key(p * negf)
                    return carry

                lax.fori_loop(0, N // L, key_step, jnp.int32(0))

                def bit_step(bi, prefix):
                    b = jnp.int32(31) - bi
                    trial = prefix | (jnp.int32(1) << b)
                    cand = trial ^ MSB

                    def cnt_step(i, acc):
                        kv = keys_v[pl.ds(i * L, L)]
                        return acc + jnp.where(kv >= cand, jnp.int32(1),
                                               jnp.int32(0))

                    acc = lax.fori_loop(0, N // L, cnt_step,
                                        jnp.zeros((L,), jnp.int32))
                    cnt_ge = jnp.sum(acc)
                    return jnp.where(cnt_ge >= k, trial, prefix)

                prefix = lax.fori_loop(0, 32, bit_step, jnp.int32(0))
                t = prefix ^ MSB

                def fin_step(i, carry):
                    acc_c, acc_s = carry
                    kv = keys_v[pl.ds(i * L, L)]
                    m = kv > t
                    sq = _unkey_sq(kv)
                    acc_c = acc_c + jnp.where(m, jnp.int32(1), jnp.int32(0))
                    acc_s = acc_s + jnp.where(m, sq, jnp.float32(0.0))
                    return acc_c, acc_s

                acc_c, acc_s = lax.fori_loop(
                    0, N // L, fin_step,
                    (jnp.zeros((L,), jnp.int32), zf))
                cnt_gt = jnp.sum(acc_c)
                sum_gt = jnp.sum(acc_s)
                tsq = jnp.max(_unkey_sq(jnp.full((L,), t, jnp.int32)))
                rem = k - cnt_gt
                tie = jnp.where(rem > 0, rem.astype(jnp.float32) * tsq,
                                jnp.float32(0.0))
                return sum_gt + tie

            return per_stack(fp0_v) + per_stack(fp1_v)

        nopt_raw = lax.cond(num_pos_i * 3 >= N, fast_nopt, slow_nopt)

        half = jnp.float32(0.5)
        iota = lax.broadcasted_iota(jnp.int32, (L,), 0)
        numer = jnp.where(
            iota == 0, pt_raw + nopt_raw,
            jnp.where(iota == 1, pt_raw,
                      jnp.where(iota == 2, nopt_raw, jnp.float32(0.0))))
        res = numer * half / jnp.full((L,), npd, jnp.float32)
        res_v[...] = res

        @pl.when(c == 0)
        def _write():
            pltpu.sync_copy(res_v, out_hbm)


def kernel(out_ct_s0, out_ct_s1, batch_ct):
    p0 = out_ct_s0.reshape(-1)
    p1 = out_ct_s1.reshape(-1)
    gt = batch_ct.reshape(-1)
    out = _sc_loss(p0, p1, gt)
    return (out[0], out[1], out[2])


# speedup vs baseline: 1.0046x; 1.0046x over previous
import functools

import jax
import jax.numpy as jnp
from jax import lax
from jax.experimental import pallas as pl
from jax.experimental.pallas import tpu as pltpu
from jax.experimental.pallas import tpu_sc as plsc

N = 16384
L = 16
NS = 16
CHUNK = N // NS
IMASK = 0x7FFFFFFF
MSB = -(2**31)


def _f32key(x):
    i = plsc.bitcast(x, jnp.int32)
    return jnp.where(i >= 0, i, i ^ IMASK)


def _unkey_sq(kv):
    iv = jnp.where(kv >= 0, kv, kv ^ IMASK)
    xv = plsc.bitcast(iv, jnp.float32)
    return xv * xv


@functools.partial(
    pl.kernel,
    out_type=jax.ShapeDtypeStruct((L,), jnp.float32),
    mesh=plsc.VectorSubcoreMesh(core_axis_name="c", subcore_axis_name="s",
                                num_cores=1),
    compiler_params=pltpu.CompilerParams(needs_layout_passes=False,
                                         skip_device_barrier=True),
    scratch_types=[
        pltpu.VMEM((CHUNK,), jnp.float32),
        pltpu.VMEM((CHUNK,), jnp.float32),
        pltpu.VMEM((CHUNK,), jnp.float32),
        pltpu.VMEM((8 * L,), jnp.float32),
        pltpu.VMEM_SHARED((NS * 8 * L,), jnp.float32),
        pltpu.VMEM((NS * 8 * L,), jnp.float32),
        pltpu.VMEM((N,), jnp.float32),
        pltpu.VMEM((N,), jnp.float32),
        pltpu.VMEM((N,), jnp.float32),
        pltpu.VMEM((N,), jnp.int32),
        pltpu.VMEM((L,), jnp.float32),
        pltpu.SemaphoreType.DMA,
        pltpu.SemaphoreType.DMA,
        pltpu.SemaphoreType.DMA,
    ],
)
def _sc_loss(p0_hbm, p1_hbm, gt_hbm, out_hbm,
             gt_v, p0_v, p1_v, part_v, part_sh, red_v,
             fgt_v, fp0_v, fp1_v, keys_v, res_v, sem0, sem1, sem2):
    c = lax.axis_index("c")
    s = lax.axis_index("s")
    base = s * CHUNK

    cp0 = pltpu.async_copy(gt_hbm.at[pl.ds(base, CHUNK)], gt_v, sem0)
    cp1 = pltpu.async_copy(p0_hbm.at[pl.ds(base, CHUNK)], p0_v, sem1)
    cp2 = pltpu.async_copy(p1_hbm.at[pl.ds(base, CHUNK)], p1_v, sem2)
    cp0.wait()
    cp1.wait()
    cp2.wait()

    zf = jnp.zeros((L,), jnp.float32)

    def p1_step(i, carry):
        cnt, pt0, pt1, sq0, sq1 = carry
        for u in range(4):
            off = (i * 4 + u) * L
            g = gt_v[pl.ds(off, L)]
            a = p0_v[pl.ds(off, L)]
            b = p1_v[pl.ds(off, L)]
            posf = jnp.where(g == 1.0, jnp.float32(1.0), jnp.float32(0.0))
            negf = jnp.float32(1.0) - posf
            da = a - jnp.float32(1.0)
            db = b - jnp.float32(1.0)
            na = a * negf
            nb = b * negf
            cnt = cnt + posf
            pt0 = pt0 + da * da * posf
            pt1 = pt1 + db * db * posf
            sq0 = sq0 + na * na
            sq1 = sq1 + nb * nb
        return (cnt, pt0, pt1, sq0, sq1)

    cnt, pt0, pt1, sq0, sq1 = lax.fori_loop(
        0, CHUNK // L // 4, p1_step, (zf, zf, zf, zf, zf))

    part_v[pl.ds(0 * L, L)] = cnt
    part_v[pl.ds(1 * L, L)] = pt0
    part_v[pl.ds(2 * L, L)] = pt1
    part_v[pl.ds(3 * L, L)] = sq0
    part_v[pl.ds(4 * L, L)] = sq1
    pltpu.sync_copy(part_v, part_sh.at[pl.ds(s * 8 * L, 8 * L)])
    plsc.subcore_barrier()

    @pl.when(s == 0)
    def _phase2():
        pltpu.sync_copy(part_sh, red_v)

        def red_step(i, carry):
            tcnt, tpt, tsq0, tsq1 = carry
            row = i * 8 * L
            tcnt = tcnt + red_v[pl.ds(row + 0 * L, L)]
            tpt = (tpt + red_v[pl.ds(row + 1 * L, L)]
                   + red_v[pl.ds(row + 2 * L, L)])
            tsq0 = tsq0 + red_v[pl.ds(row + 3 * L, L)]
            tsq1 = tsq1 + red_v[pl.ds(row + 4 * L, L)]
            return tcnt, tpt, tsq0, tsq1

        tcnt, tpt, tsq0, tsq1 = lax.fori_loop(
            0, NS, red_step, (zf, zf, zf, zf))

        num_pos = jnp.sum(tcnt)
        num_pos_i = num_pos.astype(jnp.int32)
        k = jnp.minimum(num_pos_i * 3, N)
        npd = jnp.maximum(num_pos, jnp.float32(1.0))
        pt_raw = jnp.sum(tpt)

        def fast_nopt():
            return jnp.sum(tsq0) + jnp.sum(tsq1)

        def slow_nopt():
            pltpu.sync_copy(gt_hbm.at[pl.ds(0, N)], fgt_v)
            pltpu.sync_copy(p0_hbm.at[pl.ds(0, N)], fp0_v)
            pltpu.sync_copy(p1_hbm.at[pl.ds(0, N)], fp1_v)

            def per_stack(pred_ref):
                def key_step(i, carry):
                    g = fgt_v[pl.ds(i * L, L)]
                    p = pred_ref[pl.ds(i * L, L)]
                    negf = jnp.where(g == 1.0, jnp.float32(0.0),
                                     jnp.float32(1.0))
                    keys_v[pl.ds(i * L, L)] = _f32key(p * negf)
                    return carry

                lax.fori_loop(0, N // L, key_step, jnp.int32(0))

                def bit_step(bi, prefix):
                    b = jnp.int32(31) - bi
                    trial = prefix | (jnp.int32(1) << b)
                    cand = trial ^ MSB

                    def cnt_step(i, acc):
                        kv = keys_v[pl.ds(i * L, L)]
                        return acc + jnp.where(kv >= cand, jnp.int32(1),
                                               jnp.int32(0))

                    acc = lax.fori_loop(0, N // L, cnt_step,
                                        jnp.zeros((L,), jnp.int32))
                    cnt_ge = jnp.sum(acc)
                    return jnp.where(cnt_ge >= k, trial, prefix)

                prefix = lax.fori_loop(0, 32, bit_step, jnp.int32(0))
                t = prefix ^ MSB

                def fin_step(i, carry):
                    acc_c, acc_s = carry
                    kv = keys_v[pl.ds(i * L, L)]
                    m = kv > t
                    sq = _unkey_sq(kv)
                    acc_c = acc_c + jnp.where(m, jnp.int32(1), jnp.int32(0))
                    acc_s = acc_s + jnp.where(m, sq, jnp.float32(0.0))
                    return acc_c, acc_s

                acc_c, acc_s = lax.fori_loop(
                    0, N // L, fin_step,
                    (jnp.zeros((L,), jnp.int32), zf))
                cnt_gt = jnp.sum(acc_c)
                sum_gt = jnp.sum(acc_s)
                tsq = jnp.max(_unkey_sq(jnp.full((L,), t, jnp.int32)))
                rem = k - cnt_gt
                tie = jnp.where(rem > 0, rem.astype(jnp.float32) * tsq,
                                jnp.float32(0.0))
                return sum_gt + tie

            return per_stack(fp0_v) + per_stack(fp1_v)

        nopt_raw = lax.cond(num_pos_i * 3 >= N, fast_nopt, slow_nopt)

        half = jnp.float32(0.5)
        iota = lax.broadcasted_iota(jnp.int32, (L,), 0)
        numer = jnp.where(
            iota == 0, pt_raw + nopt_raw,
            jnp.where(iota == 1, pt_raw,
                      jnp.where(iota == 2, nopt_raw, jnp.float32(0.0))))
        res = numer * half / jnp.full((L,), npd, jnp.float32)
        res_v[...] = res

        @pl.when(c == 0)
        def _write():
            pltpu.sync_copy(res_v, out_hbm)


def kernel(out_ct_s0, out_ct_s1, batch_ct):
    p0 = out_ct_s0.reshape(-1)
    p1 = out_ct_s1.reshape(-1)
    gt = batch_ct.reshape(-1)
    out = _sc_loss(p0, p1, gt)
    return (out[0], out[1], out[2])


# speedup vs baseline: 1.0273x; 1.0225x over previous
import functools

import jax
import jax.numpy as jnp
from jax import lax
from jax.experimental import pallas as pl
from jax.experimental.pallas import tpu as pltpu
from jax.experimental.pallas import tpu_sc as plsc

N = 16384
L = 16
NS = 16
CHUNK = N // NS
IMASK = 0x7FFFFFFF
MSB = -(2**31)


def _f32key(x):
    i = plsc.bitcast(x, jnp.int32)
    return jnp.where(i >= 0, i, i ^ IMASK)


def _unkey_sq(kv):
    iv = jnp.where(kv >= 0, kv, kv ^ IMASK)
    xv = plsc.bitcast(iv, jnp.float32)
    return xv * xv


@functools.partial(
    pl.kernel,
    out_type=(jax.ShapeDtypeStruct((1,), jnp.float32),
              jax.ShapeDtypeStruct((1,), jnp.float32),
              jax.ShapeDtypeStruct((1,), jnp.float32)),
    mesh=plsc.VectorSubcoreMesh(core_axis_name="c", subcore_axis_name="s",
                                num_cores=1),
    compiler_params=pltpu.CompilerParams(needs_layout_passes=False,
                                         skip_device_barrier=True),
    scratch_types=[
        pltpu.VMEM((CHUNK,), jnp.float32),
        pltpu.VMEM((CHUNK,), jnp.float32),
        pltpu.VMEM((CHUNK,), jnp.float32),
        pltpu.VMEM((8 * L,), jnp.float32),
        pltpu.VMEM_SHARED((NS * 8 * L,), jnp.float32),
        pltpu.VMEM((NS * 8 * L,), jnp.float32),
        pltpu.VMEM((N,), jnp.float32),
        pltpu.VMEM((N,), jnp.float32),
        pltpu.VMEM((N,), jnp.float32),
        pltpu.VMEM((N,), jnp.int32),
        pltpu.VMEM((2 * L,), jnp.float32),
        pltpu.SemaphoreType.DMA,
        pltpu.SemaphoreType.DMA,
        pltpu.SemaphoreType.DMA,
    ],
)
def _sc_loss(p0_hbm, p1_hbm, gt_hbm, out_loss, out_pt, out_nopt,
             gt_v, p0_v, p1_v, part_v, part_sh, red_v,
             fgt_v, fp0_v, fp1_v, keys_v, res_v, sem0, sem1, sem2):
    c = lax.axis_index("c")
    s = lax.axis_index("s")
    base = s * CHUNK

    cp0 = pltpu.async_copy(gt_hbm.at[pl.ds(base, CHUNK)], gt_v, sem0)
    cp1 = pltpu.async_copy(p0_hbm.at[pl.ds(base, CHUNK)], p0_v, sem1)
    cp2 = pltpu.async_copy(p1_hbm.at[pl.ds(base, CHUNK)], p1_v, sem2)
    cp0.wait()
    cp1.wait()
    cp2.wait()

    zf = jnp.zeros((L,), jnp.float32)

    def p1_step(i, carry):
        cnt, pt0, pt1, sq0, sq1 = carry
        for u in range(4):
            off = (i * 4 + u) * L
            g = gt_v[pl.ds(off, L)]
            a = p0_v[pl.ds(off, L)]
            b = p1_v[pl.ds(off, L)]
            posf = jnp.where(g == 1.0, jnp.float32(1.0), jnp.float32(0.0))
            negf = jnp.float32(1.0) - posf
            da = a - jnp.float32(1.0)
            db = b - jnp.float32(1.0)
            na = a * negf
            nb = b * negf
            cnt = cnt + posf
            pt0 = pt0 + da * da * posf
            pt1 = pt1 + db * db * posf
            sq0 = sq0 + na * na
            sq1 = sq1 + nb * nb
        return (cnt, pt0, pt1, sq0, sq1)

    cnt, pt0, pt1, sq0, sq1 = lax.fori_loop(
        0, CHUNK // L // 4, p1_step, (zf, zf, zf, zf, zf))

    part_v[pl.ds(0 * L, L)] = cnt
    part_v[pl.ds(1 * L, L)] = pt0
    part_v[pl.ds(2 * L, L)] = pt1
    part_v[pl.ds(3 * L, L)] = sq0
    part_v[pl.ds(4 * L, L)] = sq1
    pltpu.sync_copy(part_v, part_sh.at[pl.ds(s * 8 * L, 8 * L)])
    plsc.subcore_barrier()

    @pl.when(s == 0)
    def _phase2():
        pltpu.sync_copy(part_sh, red_v)

        def red_step(i, carry):
            tcnt, tpt, tsq0, tsq1 = carry
            row = i * 8 * L
            tcnt = tcnt + red_v[pl.ds(row + 0 * L, L)]
            tpt = (tpt + red_v[pl.ds(row + 1 * L, L)]
                   + red_v[pl.ds(row + 2 * L, L)])
            tsq0 = tsq0 + red_v[pl.ds(row + 3 * L, L)]
            tsq1 = tsq1 + red_v[pl.ds(row + 4 * L, L)]
            return tcnt, tpt, tsq0, tsq1

        tcnt, tpt, tsq0, tsq1 = lax.fori_loop(
            0, NS, red_step, (zf, zf, zf, zf))

        num_pos = jnp.sum(tcnt)
        num_pos_i = num_pos.astype(jnp.int32)
        k = jnp.minimum(num_pos_i * 3, N)
        npd = jnp.maximum(num_pos, jnp.float32(1.0))
        pt_raw = jnp.sum(tpt)

        def fast_nopt():
            return jnp.sum(tsq0) + jnp.sum(tsq1)

        def slow_nopt():
            pltpu.sync_copy(gt_hbm.at[pl.ds(0, N)], fgt_v)
            pltpu.sync_copy(p0_hbm.at[pl.ds(0, N)], fp0_v)
            pltpu.sync_copy(p1_hbm.at[pl.ds(0, N)], fp1_v)

            def per_stack(pred_ref):
                def key_step(i, carry):
                    g = fgt_v[pl.ds(i * L, L)]
                    p = pred_ref[pl.ds(i * L, L)]
                    negf = jnp.where(g == 1.0, jnp.float32(0.0),
                                     jnp.float32(1.0))
                    keys_v[pl.ds(i * L, L)] = _f32key(p * negf)
                    return carry

                lax.fori_loop(0, N // L, key_step, jnp.int32(0))

                def bit_step(bi, prefix):
                    b = jnp.int32(31) - bi
                    trial = prefix | (jnp.int32(1) << b)
                    cand = trial ^ MSB

                    def cnt_step(i, acc):
                        kv = keys_v[pl.ds(i * L, L)]
                        return acc + jnp.where(kv >= cand, jnp.int32(1),
                                               jnp.int32(0))

                    acc = lax.fori_loop(0, N // L, cnt_step,
                                        jnp.zeros((L,), jnp.int32))
                    cnt_ge = jnp.sum(acc)
                    return jnp.where(cnt_ge >= k, trial, prefix)

                prefix = lax.fori_loop(0, 32, bit_step, jnp.int32(0))
                t = prefix ^ MSB

                def fin_step(i, carry):
                    acc_c, acc_s = carry
                    kv = keys_v[pl.ds(i * L, L)]
                    m = kv > t
                    sq = _unkey_sq(kv)
                    acc_c = acc_c + jnp.where(m, jnp.int32(1), jnp.int32(0))
                    acc_s = acc_s + jnp.where(m, sq, jnp.float32(0.0))
                    return acc_c, acc_s

                acc_c, acc_s = lax.fori_loop(
                    0, N // L, fin_step,
                    (jnp.zeros((L,), jnp.int32), zf))
                cnt_gt = jnp.sum(acc_c)
                sum_gt = jnp.sum(acc_s)
                tsq = jnp.max(_unkey_sq(jnp.full((L,), t, jnp.int32)))
                rem = k - cnt_gt
                tie = jnp.where(rem > 0, rem.astype(jnp.float32) * tsq,
                                jnp.float32(0.0))
                return sum_gt + tie

            return per_stack(fp0_v) + per_stack(fp1_v)

        nopt_raw = lax.cond(num_pos_i * 3 >= N, fast_nopt, slow_nopt)

        half = jnp.float32(0.5)
        iota = lax.broadcasted_iota(jnp.int32, (L,), 0)
        npd_vec = jnp.full((L,), npd, jnp.float32)
        numer0 = jnp.where(
            iota == 0, pt_raw + nopt_raw,
            jnp.where(iota == 8, pt_raw, jnp.float32(0.0)))
        numer1 = jnp.where(iota == 0, nopt_raw, jnp.float32(0.0))
        res_v[pl.ds(0, L)] = numer0 * half / npd_vec
        res_v[pl.ds(L, L)] = numer1 * half / npd_vec

        @pl.when(c == 0)
        def _write():
            pltpu.sync_copy(res_v.at[pl.ds(0, 1)], out_loss)
            pltpu.sync_copy(res_v.at[pl.ds(8, 1)], out_pt)
            pltpu.sync_copy(res_v.at[pl.ds(16, 1)], out_nopt)


def kernel(out_ct_s0, out_ct_s1, batch_ct):
    p0 = out_ct_s0.reshape(-1)
    p1 = out_ct_s1.reshape(-1)
    gt = batch_ct.reshape(-1)
    loss, ct_pt, ct_nopt = _sc_loss(p0, p1, gt)
    return (loss.reshape(()), ct_pt.reshape(()), ct_nopt.reshape(()))


# speedup vs baseline: 1.0299x; 1.0026x over previous
import functools

import jax
import jax.numpy as jnp
from jax import lax
from jax.experimental import pallas as pl
from jax.experimental.pallas import tpu as pltpu
from jax.experimental.pallas import tpu_sc as plsc

N = 16384
L = 16
NS = 16
CHUNK = N // NS
IMASK = 0x7FFFFFFF
MSB = -(2**31)


def _f32key(x):
    i = plsc.bitcast(x, jnp.int32)
    return jnp.where(i >= 0, i, i ^ IMASK)


def _unkey_sq(kv):
    iv = jnp.where(kv >= 0, kv, kv ^ IMASK)
    xv = plsc.bitcast(iv, jnp.float32)
    return xv * xv


@functools.partial(
    pl.kernel,
    out_type=(jax.ShapeDtypeStruct((1,), jnp.float32),
              jax.ShapeDtypeStruct((1,), jnp.float32),
              jax.ShapeDtypeStruct((1,), jnp.float32)),
    mesh=plsc.VectorSubcoreMesh(core_axis_name="c", subcore_axis_name="s",
                                num_cores=1),
    compiler_params=pltpu.CompilerParams(needs_layout_passes=False,
                                         skip_device_barrier=True),
    scratch_types=[
        pltpu.VMEM((CHUNK,), jnp.float32),
        pltpu.VMEM((CHUNK,), jnp.float32),
        pltpu.VMEM((CHUNK,), jnp.float32),
        pltpu.VMEM((5 * L,), jnp.float32),
        pltpu.VMEM_SHARED((NS * 5 * L,), jnp.float32),
        pltpu.VMEM((NS * 5 * L,), jnp.float32),
        pltpu.VMEM((N,), jnp.float32),
        pltpu.VMEM((N,), jnp.float32),
        pltpu.VMEM((N,), jnp.float32),
        pltpu.VMEM((N,), jnp.int32),
        pltpu.VMEM((2 * L,), jnp.float32),
        pltpu.SemaphoreType.DMA,
        pltpu.SemaphoreType.DMA,
        pltpu.SemaphoreType.DMA,
    ],
)
def _sc_loss(p0_hbm, p1_hbm, gt_hbm, out_loss, out_pt, out_nopt,
             gt_v, p0_v, p1_v, part_v, part_sh, red_v,
             fgt_v, fp0_v, fp1_v, keys_v, res_v, sem0, sem1, sem2):
    c = lax.axis_index("c")
    s = lax.axis_index("s")
    base = s * CHUNK

    cp0 = pltpu.async_copy(gt_hbm.at[pl.ds(base, CHUNK)], gt_v, sem0)
    cp1 = pltpu.async_copy(p0_hbm.at[pl.ds(base, CHUNK)], p0_v, sem1)
    cp2 = pltpu.async_copy(p1_hbm.at[pl.ds(base, CHUNK)], p1_v, sem2)
    cp0.wait()
    cp1.wait()
    cp2.wait()

    zf = jnp.zeros((L,), jnp.float32)

    def p1_step(i, carry):
        cnt, pt0, pt1, sq0, sq1 = carry
        for u in range(4):
            off = (i * 4 + u) * L
            g = gt_v[pl.ds(off, L)]
            a = p0_v[pl.ds(off, L)]
            b = p1_v[pl.ds(off, L)]
            posf = jnp.where(g == 1.0, jnp.float32(1.0), jnp.float32(0.0))
            negf = jnp.float32(1.0) - posf
            da = a - jnp.float32(1.0)
            db = b - jnp.float32(1.0)
            na = a * negf
            nb = b * negf
            cnt = cnt + posf
            pt0 = pt0 + da * da * posf
            pt1 = pt1 + db * db * posf
            sq0 = sq0 + na * na
            sq1 = sq1 + nb * nb
        return (cnt, pt0, pt1, sq0, sq1)

    cnt, pt0, pt1, sq0, sq1 = lax.fori_loop(
        0, CHUNK // L // 4, p1_step, (zf, zf, zf, zf, zf))

    part_v[pl.ds(0 * L, L)] = cnt
    part_v[pl.ds(1 * L, L)] = pt0
    part_v[pl.ds(2 * L, L)] = pt1
    part_v[pl.ds(3 * L, L)] = sq0
    part_v[pl.ds(4 * L, L)] = sq1
    pltpu.sync_copy(part_v, part_sh.at[pl.ds(s * 5 * L, 5 * L)])
    plsc.subcore_barrier()

    @pl.when(s == 0)
    def _phase2():
        pltpu.sync_copy(part_sh, red_v)

        def red_step(i, carry):
            tcnt, tpt, tsq0, tsq1 = carry
            row = i * 5 * L
            tcnt = tcnt + red_v[pl.ds(row + 0 * L, L)]
            tpt = (tpt + red_v[pl.ds(row + 1 * L, L)]
                   + red_v[pl.ds(row + 2 * L, L)])
            tsq0 = tsq0 + red_v[pl.ds(row + 3 * L, L)]
            tsq1 = tsq1 + red_v[pl.ds(row + 4 * L, L)]
            return tcnt, tpt, tsq0, tsq1

        tcnt, tpt, tsq0, tsq1 = lax.fori_loop(
            0, NS, red_step, (zf, zf, zf, zf))

        num_pos = jnp.sum(tcnt)
        num_pos_i = num_pos.astype(jnp.int32)
        k = jnp.minimum(num_pos_i * 3, N)
        npd = jnp.maximum(num_pos, jnp.float32(1.0))
        pt_raw = jnp.sum(tpt)

        def fast_nopt():
            return jnp.sum(tsq0) + jnp.sum(tsq1)

        def slow_nopt():
            pltpu.sync_copy(gt_hbm.at[pl.ds(0, N)], fgt_v)
            pltpu.sync_copy(p0_hbm.at[pl.ds(0, N)], fp0_v)
            pltpu.sync_copy(p1_hbm.at[pl.ds(0, N)], fp1_v)

            def per_stack(pred_ref):
                def key_step(i, carry):
                    g = fgt_v[pl.ds(i * L, L)]
                    p = pred_ref[pl.ds(i * L, L)]
                    negf = jnp.where(g == 1.0, jnp.float32(0.0),
                                     jnp.float32(1.0))
                    keys_v[pl.ds(i * L, L)] = _f32key(p * negf)
                    return carry

                lax.fori_loop(0, N // L, key_step, jnp.int32(0))

                def bit_step(bi, prefix):
                    b = jnp.int32(31) - bi
                    trial = prefix | (jnp.int32(1) << b)
                    cand = trial ^ MSB

                    def cnt_step(i, acc):
                        kv = keys_v[pl.ds(i * L, L)]
                        return acc + jnp.where(kv >= cand, jnp.int32(1),
                                               jnp.int32(0))

                    acc = lax.fori_loop(0, N // L, cnt_step,
                                        jnp.zeros((L,), jnp.int32))
                    cnt_ge = jnp.sum(acc)
                    return jnp.where(cnt_ge >= k, trial, prefix)

                prefix = lax.fori_loop(0, 32, bit_step, jnp.int32(0))
                t = prefix ^ MSB

                def fin_step(i, carry):
                    acc_c, acc_s = carry
                    kv = keys_v[pl.ds(i * L, L)]
                    m = kv > t
                    sq = _unkey_sq(kv)
                    acc_c = acc_c + jnp.where(m, jnp.int32(1), jnp.int32(0))
                    acc_s = acc_s + jnp.where(m, sq, jnp.float32(0.0))
                    return acc_c, acc_s

                acc_c, acc_s = lax.fori_loop(
                    0, N // L, fin_step,
                    (jnp.zeros((L,), jnp.int32), zf))
                cnt_gt = jnp.sum(acc_c)
                sum_gt = jnp.sum(acc_s)
                tsq = jnp.max(_unkey_sq(jnp.full((L,), t, jnp.int32)))
                rem = k - cnt_gt
                tie = jnp.where(rem > 0, rem.astype(jnp.float32) * tsq,
                                jnp.float32(0.0))
                return sum_gt + tie

            return per_stack(fp0_v) + per_stack(fp1_v)

        nopt_raw = lax.cond(num_pos_i * 3 >= N, fast_nopt, slow_nopt)

        half = jnp.float32(0.5)
        iota = lax.broadcasted_iota(jnp.int32, (L,), 0)
        npd_vec = jnp.full((L,), npd, jnp.float32)
        numer0 = jnp.where(
            iota == 0, pt_raw + nopt_raw,
            jnp.where(iota == 8, pt_raw, jnp.float32(0.0)))
        numer1 = jnp.where(iota == 0, nopt_raw, jnp.float32(0.0))
        res_v[pl.ds(0, L)] = numer0 * half / npd_vec
        res_v[pl.ds(L, L)] = numer1 * half / npd_vec

        @pl.when(c == 0)
        def _write():
            pltpu.sync_copy(res_v.at[pl.ds(0, 1)], out_loss)
            pltpu.sync_copy(res_v.at[pl.ds(8, 1)], out_pt)
            pltpu.sync_copy(res_v.at[pl.ds(16, 1)], out_nopt)


def kernel(out_ct_s0, out_ct_s1, batch_ct):
    p0 = out_ct_s0.reshape(-1)
    p1 = out_ct_s1.reshape(-1)
    gt = batch_ct.reshape(-1)
    loss, ct_pt, ct_nopt = _sc_loss(p0, p1, gt)
    return (loss.reshape(()), ct_pt.reshape(()), ct_nopt.reshape(()))
